# Initial kernel scaffold; baseline (speedup 1.0000x reference)
import jax
import jax.numpy as jnp
from jax.experimental import pallas as pl


def kernel(flow, inv_depth):
    raise NotImplementedError("write your pallas kernel here")


# speedup vs baseline: 44.3355x; 44.3355x over previous
import functools

import jax
import jax.numpy as jnp
from jax import lax
from jax.experimental import pallas as pl
from jax.experimental.pallas import tpu as pltpu
from jax.experimental.pallas import tpu_sc as plsc

_B, _H, _W = 8, 512, 512
_HW = _H * _W
_NC, _NS, _L = 2, 16, 16
_CH = _HW // _NS
_GROUP = 1024
_NG = _CH // _GROUP
_T = _GROUP // _L
_BPC = _B // _NC
_RC = 12582912.0
_EPS = 1e-7


def _body(flow_ref, idep_ref, out_ref, accx, accy, accw,
          fxb, fyb, idb, zb, wxb, wyb, wwb, ixb, sem):
    c = lax.axis_index("c")
    s = lax.axis_index("s")
    base = s * _CH
    iof = lax.iota(jnp.int32, _L).astype(jnp.float32)
    zeros16 = jnp.zeros((_L,), jnp.float32)

    def zinit(i, carry):
        zb[pl.ds(i * _L, _L)] = zeros16
        return carry

    lax.fori_loop(0, _CH // _L, zinit, 0)

    for k in range(_BPC):
        b = c * _BPC + k
        pltpu.sync_copy(zb, accx.at[pl.ds(base, _CH)])
        pltpu.sync_copy(zb, accy.at[pl.ds(base, _CH)])
        pltpu.sync_copy(zb, accw.at[pl.ds(base, _CH)])
        pltpu.sync_copy(flow_ref.at[b, 0, pl.ds(base, _CH)], fxb)
        pltpu.sync_copy(flow_ref.at[b, 1, pl.ds(base, _CH)], fyb)
        pltpu.sync_copy(idep_ref.at[b, pl.ds(base, _CH)], idb)
        plsc.subcore_barrier()

        def group(g, carry):
            for t in range(_T):
                local = g * _GROUP + t * _L
                fx = fxb[pl.ds(local, _L)]
                fy = fyb[pl.ds(local, _L)]
                dv = idb[pl.ds(local, _L)]
                cx = iof + float((t * _L) % _W)
                yrow = s * (_CH // _W) + g * (_GROUP // _W) + (t * _L) // _W
                rx = (cx - fx + _RC) - _RC
                ry = (yrow.astype(jnp.float32) - fy + _RC) - _RC
                ix = rx.astype(jnp.int32)
                iy = ry.astype(jnp.int32)
                inr = ((ix.astype(jnp.uint32) < jnp.uint32(_W))
                       & (iy.astype(jnp.uint32) < jnp.uint32(_H)))
                w = jnp.where(inr, dv, jnp.float32(0.0))
                idx = jnp.where(inr, iy * _W + ix, 0)
                o = t * _L
                wxb[pl.ds(o, _L)] = fx * w
                wyb[pl.ds(o, _L)] = fy * w
                wwb[pl.ds(o, _L)] = w
                ixb[pl.ds(o, _L)] = idx
            d1 = pltpu.async_copy(wxb, accx.at[ixb], sem, add=True)
            d2 = pltpu.async_copy(wyb, accy.at[ixb], sem, add=True)
            d3 = pltpu.async_copy(wwb, accw.at[ixb], sem, add=True)
            d1.wait()
            d2.wait()
            d3.wait()
            return carry

        lax.fori_loop(0, _NG, group, 0)
        plsc.subcore_barrier()

        pltpu.sync_copy(accx.at[pl.ds(base, _CH)], fxb)
        pltpu.sync_copy(accy.at[pl.ds(base, _CH)], fyb)
        pltpu.sync_copy(accw.at[pl.ds(base, _CH)], idb)

        def fin(i, carry):
            sl = pl.ds(i * _L, _L)
            ax = fxb[sl]
            ay = fyb[sl]
            aw = idb[sl]
            inv = jnp.float32(1.0) / (aw + _EPS)
            m = ax != jnp.float32(0.0)
            fxb[sl] = jnp.where(m, ax * inv, jnp.float32(0.0))
            fyb[sl] = jnp.where(m, ay * inv, jnp.float32(0.0))
            return carry

        lax.fori_loop(0, _CH // _L, fin, 0)
        pltpu.sync_copy(fxb, out_ref.at[b, 0, pl.ds(base, _CH)])
        pltpu.sync_copy(fyb, out_ref.at[b, 1, pl.ds(base, _CH)])


def kernel(flow, inv_depth):
    flow_r = flow.reshape(_B, 2, _HW)
    idep_r = inv_depth.reshape(_B, _HW)
    mesh = plsc.VectorSubcoreMesh(core_axis_name="c", subcore_axis_name="s",
                                  num_cores=_NC, num_subcores=_NS)
    kfn = pl.kernel(
        _body,
        out_type=jax.ShapeDtypeStruct((_B, 2, _HW), jnp.float32),
        mesh=mesh,
        scratch_types=[
            pltpu.VMEM_SHARED((_HW,), jnp.float32),
            pltpu.VMEM_SHARED((_HW,), jnp.float32),
            pltpu.VMEM_SHARED((_HW,), jnp.float32),
            pltpu.VMEM((_CH,), jnp.float32),
            pltpu.VMEM((_CH,), jnp.float32),
            pltpu.VMEM((_CH,), jnp.float32),
            pltpu.VMEM((_CH,), jnp.float32),
            pltpu.VMEM((_GROUP,), jnp.float32),
            pltpu.VMEM((_GROUP,), jnp.float32),
            pltpu.VMEM((_GROUP,), jnp.float32),
            pltpu.VMEM((_GROUP,), jnp.int32),
            pltpu.SemaphoreType.DMA,
        ],
    )
    out = kfn(flow_r, idep_r)
    return out.reshape(_B, 2, _H, _W)


# speedup vs baseline: 54.2664x; 1.2240x over previous
import functools

import jax
import jax.numpy as jnp
from jax import lax
from jax.experimental import pallas as pl
from jax.experimental.pallas import tpu as pltpu
from jax.experimental.pallas import tpu_sc as plsc

_B, _H, _W = 8, 512, 512
_HW = _H * _W
_NC, _NS, _L = 2, 16, 16
_CH = _HW // _NS
_GROUP = 1024
_NG = _CH // _GROUP
_T = _GROUP // _L
_BPC = _B // _NC
_RC = 12582912.0
_EPS = 1e-7


def _body(flow_ref, idep_ref, out_ref, accx, accy, accw,
          fxb, fyb, idb, zb,
          wxb0, wyb0, wwb0, ixb0, wxb1, wyb1, wwb1, ixb1, sem0, sem1):
    c = lax.axis_index("c")
    s = lax.axis_index("s")
    base = s * _CH
    iof = lax.iota(jnp.int32, _L).astype(jnp.float32)
    zeros16 = jnp.zeros((_L,), jnp.float32)
    rings = ((wxb0, wyb0, wwb0, ixb0, sem0), (wxb1, wyb1, wwb1, ixb1, sem1))
    accs = (accx, accy, accw)
    pending = [[], []]

    def zinit(i, carry):
        zb[pl.ds(i * _L, _L)] = zeros16
        return carry

    lax.fori_loop(0, _CH // _L, zinit, 0)

    for k in range(_BPC):
        b = c * _BPC + k
        pltpu.sync_copy(zb, accx.at[pl.ds(base, _CH)])
        pltpu.sync_copy(zb, accy.at[pl.ds(base, _CH)])
        pltpu.sync_copy(zb, accw.at[pl.ds(base, _CH)])
        pltpu.sync_copy(flow_ref.at[b, 0, pl.ds(base, _CH)], fxb)
        pltpu.sync_copy(flow_ref.at[b, 1, pl.ds(base, _CH)], fyb)
        pltpu.sync_copy(idep_ref.at[b, pl.ds(base, _CH)], idb)
        plsc.subcore_barrier()

        for g in range(_NG):
            wxb, wyb, wwb, ixb, sem = rings[g % 2]
            if g >= 2:
                for dd in pending[g % 2]:
                    dd.wait()

            def titer(t, carry, g=g, wxb=wxb, wyb=wyb, wwb=wwb, ixb=ixb):
                o = t * _L
                local = g * _GROUP + o
                fx = fxb[pl.ds(local, _L)]
                fy = fyb[pl.ds(local, _L)]
                dv = idb[pl.ds(local, _L)]
                x0 = o & (_W - 1)
                cx = iof + x0.astype(jnp.float32)
                yrow = s * (_CH // _W) + g * (_GROUP // _W) + (t >> 5)
                rx = (cx - fx + _RC) - _RC
                ry = (yrow.astype(jnp.float32) - fy + _RC) - _RC
                ix = rx.astype(jnp.int32)
                iy = ry.astype(jnp.int32)
                inr = ((ix.astype(jnp.uint32) < jnp.uint32(_W))
                       & (iy.astype(jnp.uint32) < jnp.uint32(_H)))
                w = jnp.where(inr, dv, jnp.float32(0.0))
                idx = jnp.where(inr, iy * _W + ix, 0)
                wxb[pl.ds(o, _L)] = fx * w
                wyb[pl.ds(o, _L)] = fy * w
                wwb[pl.ds(o, _L)] = w
                ixb[pl.ds(o, _L)] = idx
                return carry

            lax.fori_loop(0, _T, titer, 0)
            pending[g % 2] = [
                pltpu.async_copy(buf, acc.at[ixb], sem, add=True)
                for buf, acc in zip((wxb, wyb, wwb), accs)]

        for nb in range(2):
            for dd in pending[nb]:
                dd.wait()
            pending[nb] = []
        plsc.subcore_barrier()

        pltpu.sync_copy(accx.at[pl.ds(base, _CH)], fxb)
        pltpu.sync_copy(accy.at[pl.ds(base, _CH)], fyb)
        pltpu.sync_copy(accw.at[pl.ds(base, _CH)], idb)

        def fin(i, carry):
            sl = pl.ds(i * _L, _L)
            ax = fxb[sl]
            ay = fyb[sl]
            aw = idb[sl]
            inv = jnp.float32(1.0) / (aw + jnp.float32(_EPS))
            m = ax != jnp.float32(0.0)
            fxb[sl] = jnp.where(m, ax * inv, jnp.float32(0.0))
            fyb[sl] = jnp.where(m, ay * inv, jnp.float32(0.0))
            return carry

        lax.fori_loop(0, _CH // _L, fin, 0)
        pltpu.sync_copy(fxb, out_ref.at[b, 0, pl.ds(base, _CH)])
        pltpu.sync_copy(fyb, out_ref.at[b, 1, pl.ds(base, _CH)])


def kernel(flow, inv_depth):
    flow_r = flow.reshape(_B, 2, _HW)
    idep_r = inv_depth.reshape(_B, _HW)
    mesh = plsc.VectorSubcoreMesh(core_axis_name="c", subcore_axis_name="s",
                                  num_cores=_NC, num_subcores=_NS)
    ring_buf = [pltpu.VMEM((_GROUP,), jnp.float32)] * 3 + [
        pltpu.VMEM((_GROUP,), jnp.int32)]
    kfn = pl.kernel(
        _body,
        out_type=jax.ShapeDtypeStruct((_B, 2, _HW), jnp.float32),
        mesh=mesh,
        scratch_types=[
            pltpu.VMEM_SHARED((_HW,), jnp.float32),
            pltpu.VMEM_SHARED((_HW,), jnp.float32),
            pltpu.VMEM_SHARED((_HW,), jnp.float32),
            pltpu.VMEM((_CH,), jnp.float32),
            pltpu.VMEM((_CH,), jnp.float32),
            pltpu.VMEM((_CH,), jnp.float32),
            pltpu.VMEM((_CH,), jnp.float32),
            *ring_buf, *ring_buf,
            pltpu.SemaphoreType.DMA,
            pltpu.SemaphoreType.DMA,
        ],
    )
    out = kfn(flow_r, idep_r)
    return out.reshape(_B, 2, _H, _W)
